# Initial kernel scaffold; baseline (speedup 1.0000x reference)
import jax
import jax.numpy as jnp
from jax.experimental import pallas as pl


def kernel(x, edge_index, deg_hist, lins_W, lins_b, pre_W0, pre_b0, post_W0, post_b0, pre_W1, pre_b1, post_W1, post_b1):
    raise NotImplementedError("write your pallas kernel here")


# speedup vs baseline: 1.1396x; 1.1396x over previous
import jax
import jax.numpy as jnp
from jax.experimental import pallas as pl

_N = 10000
_E = 160000
_H = 256


def kernel(x, edge_index, deg_hist, lins_W, lins_b, pre_W0, pre_b0, post_W0, post_b0, pre_W1, pre_b1, post_W1, post_b1):
    src = edge_index[0]
    dst = edge_index[1]
    bins = jnp.arange(deg_hist.shape[0], dtype=jnp.float32)
    total = jnp.maximum(deg_hist.sum(), 1.0)
    avg_log = jnp.maximum((jnp.log(bins + 1.0) * deg_hist).sum() / total, 1e-5)

    h = x @ lins_W + lins_b
    deg = jax.ops.segment_sum(jnp.ones((_E,), jnp.float32), dst, num_segments=_N)
    degc = jnp.maximum(deg, 1.0)
    slog = jnp.log(degc + 1.0)
    amp = (slog / avg_log)[:, None]
    att = (avg_log / slog)[:, None]
    has = (deg > 0)[:, None]
    degN = deg[:, None]
    degcN = degc[:, None]

    def conv(h, preW, preb, postW, postb):
        A = h @ preW[:_H] + preb
        B = h @ preW[_H:]
        Bs = B[src]
        S = jax.ops.segment_sum(Bs, dst, num_segments=_N)
        S2 = jax.ops.segment_sum(Bs * Bs, dst, num_segments=_N)
        MX = jax.ops.segment_max(Bs, dst, num_segments=_N)
        MN = jax.ops.segment_min(Bs, dst, num_segments=_N)
        mean = (degN * A + S) / degcN
        mean_sq = (degN * A * A + 2.0 * A * S + S2) / degcN
        std = jnp.sqrt(jax.nn.relu(mean_sq - mean * mean) + 1e-5)
        mx = jnp.where(has, A + MX, 0.0)
        mn = jnp.where(has, A + MN, 0.0)
        aggs = jnp.concatenate([mean, mn, mx, std], axis=-1)
        out = (aggs @ postW[: 4 * _H]
               + amp * (aggs @ postW[4 * _H: 8 * _H])
               + att * (aggs @ postW[8 * _H:])
               + postb)
        return out

    h = jax.nn.relu(conv(h, pre_W0, pre_b0, post_W0, post_b0))
    h = conv(h, pre_W1, pre_b1, post_W1, post_b1)
    return h


# speedup vs baseline: 3.1609x; 2.7738x over previous
import functools

import jax
import jax.numpy as jnp
from jax.experimental import pallas as pl
from jax.experimental.pallas import tpu as pltpu

_VMEM_LIM = 100 * 1024 * 1024


def _mm_kern(x_ref, w_ref, b_ref, o_ref):
    o_ref[...] = (
        jnp.dot(x_ref[...], w_ref[...], preferred_element_type=jnp.float32)
        + b_ref[...]
    )


def _mm(x, w, b, bm):
    n, k = x.shape
    m = w.shape[1]
    return pl.pallas_call(
        _mm_kern,
        grid=(n // bm,),
        in_specs=[
            pl.BlockSpec((bm, k), lambda i: (i, 0)),
            pl.BlockSpec((k, m), lambda i: (0, 0)),
            pl.BlockSpec((1, m), lambda i: (0, 0)),
        ],
        out_specs=pl.BlockSpec((bm, m), lambda i: (i, 0)),
        out_shape=jax.ShapeDtypeStruct((n, m), jnp.float32),
        compiler_params=pltpu.CompilerParams(vmem_limit_bytes=_VMEM_LIM),
    )(x, w, b)


def _ab_kern(h_ref, wd_ref, ws_ref, b_ref, a_ref, bb_ref):
    h = h_ref[...]
    a_ref[...] = (
        jnp.dot(h, wd_ref[...], preferred_element_type=jnp.float32) + b_ref[...]
    )
    bb_ref[...] = jnp.dot(h, ws_ref[...], preferred_element_type=jnp.float32)


def _ab(h, wd, ws, b, bm):
    n, k = h.shape
    m = wd.shape[1]
    return pl.pallas_call(
        _ab_kern,
        grid=(n // bm,),
        in_specs=[
            pl.BlockSpec((bm, k), lambda i: (i, 0)),
            pl.BlockSpec((k, m), lambda i: (0, 0)),
            pl.BlockSpec((k, m), lambda i: (0, 0)),
            pl.BlockSpec((1, m), lambda i: (0, 0)),
        ],
        out_specs=[
            pl.BlockSpec((bm, m), lambda i: (i, 0)),
            pl.BlockSpec((bm, m), lambda i: (i, 0)),
        ],
        out_shape=[
            jax.ShapeDtypeStruct((n, m), jnp.float32),
            jax.ShapeDtypeStruct((n, m), jnp.float32),
        ],
        compiler_params=pltpu.CompilerParams(vmem_limit_bytes=_VMEM_LIM),
    )(h, wd, ws, b)


def _seg_kern(want_deg, chunk, src_ref, dst_ref, b_ref, s_ref, s2_ref, mx_ref,
              mn_ref, deg_ref):
    @pl.when(pl.program_id(0) == 0)
    def _init():
        s_ref[...] = jnp.zeros_like(s_ref)
        s2_ref[...] = jnp.zeros_like(s2_ref)
        mx_ref[...] = jnp.full_like(mx_ref, -3.0e38)
        mn_ref[...] = jnp.full_like(mn_ref, 3.0e38)
        if want_deg:
            deg_ref[...] = jnp.zeros_like(deg_ref)

    def body(e, _):
        s = src_ref[0, 0, e]
        d = dst_ref[0, 0, e]
        row = b_ref[pl.ds(s, 1), :]
        ds = pl.ds(d, 1)
        s_ref[ds, :] += row
        s2_ref[ds, :] += row * row
        mx_ref[ds, :] = jnp.maximum(mx_ref[ds, :], row)
        mn_ref[ds, :] = jnp.minimum(mn_ref[ds, :], row)
        if want_deg:
            deg_ref[ds, :] += 1.0
        return 0

    jax.lax.fori_loop(0, chunk, body, 0, unroll=4)


def _seg(b, src3, dst3, want_deg):
    n, f = b.shape
    nch, _, chunk = src3.shape
    outs = [
        jax.ShapeDtypeStruct((n, f), jnp.float32),
        jax.ShapeDtypeStruct((n, f), jnp.float32),
        jax.ShapeDtypeStruct((n, f), jnp.float32),
        jax.ShapeDtypeStruct((n, f), jnp.float32),
        jax.ShapeDtypeStruct((n, 1), jnp.float32),
    ]
    full = lambda i: (0, 0)
    res = pl.pallas_call(
        functools.partial(_seg_kern, want_deg, chunk),
        grid=(nch,),
        in_specs=[
            pl.BlockSpec((1, 1, chunk), lambda i: (i, 0, 0),
                         memory_space=pltpu.SMEM),
            pl.BlockSpec((1, 1, chunk), lambda i: (i, 0, 0),
                         memory_space=pltpu.SMEM),
            pl.BlockSpec((n, f), full),
        ],
        out_specs=[pl.BlockSpec((n, f), full)] * 4
        + [pl.BlockSpec((n, 1), full)],
        out_shape=outs,
        compiler_params=pltpu.CompilerParams(vmem_limit_bytes=_VMEM_LIM),
    )(src3, dst3, b)
    return res


def _post_kern(relu_out, a_ref, s_ref, s2_ref, mx_ref, mn_ref, deg_ref,
               hist_ref, w_ref, b_ref, o_ref):
    hist = hist_ref[...]
    nb = hist.shape[1]
    bins = jax.lax.broadcasted_iota(jnp.int32, (1, nb), 1).astype(jnp.float32)
    total = jnp.maximum(jnp.sum(hist), 1.0)
    avg_log = jnp.maximum(jnp.sum(jnp.log(bins + 1.0) * hist) / total, 1e-5)

    deg = deg_ref[...]
    degc = jnp.maximum(deg, 1.0)
    inv = 1.0 / degc
    a = a_ref[...]
    s = s_ref[...]
    s2 = s2_ref[...]
    mean = (deg * a + s) * inv
    sm = s * inv
    var = s2 * inv - sm * sm
    std = jnp.sqrt(jnp.maximum(var, 0.0) + 1e-5)
    has = deg > 0.0
    mx = jnp.where(has, a + mx_ref[...], 0.0)
    mn = jnp.where(has, a + mn_ref[...], 0.0)
    aggs = jnp.concatenate([mean, mn, mx, std], axis=-1)
    fh = aggs.shape[1]
    w = w_ref[...]
    o1 = jnp.dot(aggs, w[:fh], preferred_element_type=jnp.float32)
    o2 = jnp.dot(aggs, w[fh:2 * fh], preferred_element_type=jnp.float32)
    o3 = jnp.dot(aggs, w[2 * fh:], preferred_element_type=jnp.float32)
    slog = jnp.log(degc + 1.0)
    amp = slog / avg_log
    att = avg_log / slog
    out = o1 + amp * o2 + att * o3 + b_ref[...]
    if relu_out:
        out = jnp.maximum(out, 0.0)
    o_ref[...] = out


def _post(a, s, s2, mx, mn, deg, hist, w, b, relu_out, bm):
    n, f = a.shape
    m = w.shape[1]
    blk = lambda i: (i, 0)
    full = lambda i: (0, 0)
    return pl.pallas_call(
        functools.partial(_post_kern, relu_out),
        grid=(n // bm,),
        in_specs=[
            pl.BlockSpec((bm, f), blk),
            pl.BlockSpec((bm, f), blk),
            pl.BlockSpec((bm, f), blk),
            pl.BlockSpec((bm, f), blk),
            pl.BlockSpec((bm, f), blk),
            pl.BlockSpec((bm, 1), blk),
            pl.BlockSpec((1, hist.shape[1]), full),
            pl.BlockSpec((w.shape[0], m), full),
            pl.BlockSpec((1, m), full),
        ],
        out_specs=pl.BlockSpec((bm, m), blk),
        out_shape=jax.ShapeDtypeStruct((n, m), jnp.float32),
        compiler_params=pltpu.CompilerParams(vmem_limit_bytes=_VMEM_LIM),
    )(a, s, s2, mx, mn, deg, hist, w, b)


def kernel(x, edge_index, deg_hist, lins_W, lins_b, pre_W0, pre_b0, post_W0,
           post_b0, pre_W1, pre_b1, post_W1, post_b1):
    n, fin = x.shape
    e = edge_index.shape[1]
    hid = lins_W.shape[1]
    bm = 1000 if n % 1000 == 0 else n // 8

    nch = 20 if e % 20 == 0 else 1
    chunk = e // nch
    src3 = edge_index[0].reshape(nch, 1, chunk)
    dst3 = edge_index[1].reshape(nch, 1, chunk)
    hist2 = deg_hist.reshape(1, -1).astype(jnp.float32)
    lb = lins_b.reshape(1, -1)

    h = _mm(x, lins_W, lb, bm)

    deg = None
    for (pre_W, pre_b, post_W, post_b, relu_out) in (
        (pre_W0, pre_b0, post_W0, post_b0, True),
        (pre_W1, pre_b1, post_W1, post_b1, False),
    ):
        a, bb = _ab(h, pre_W[:hid], pre_W[hid:], pre_b.reshape(1, -1), bm)
        s, s2, mx, mn, d = _seg(bb, src3, dst3, deg is None)
        if deg is None:
            deg = d
        h = _post(a, s, s2, mx, mn, deg, hist2, post_W,
                  post_b.reshape(1, -1), relu_out, bm)
    return h
